# baseline (speedup 1.0000x reference)
import functools

import jax
import jax.numpy as jnp
from jax import lax
from jax.experimental import pallas as pl
from jax.experimental.pallas import tpu as pltpu
from jax.experimental.pallas import tpu_sc as plsc

N_NODES = 10000
N_EDGES = 160000
D_FEAT = 256
DW = D_FEAT // 2

NC, NS, L = 2, 16, 16
WPB = DW // NS
E_PAD = 163840
EC = E_PAD // NC
CH = 4096
NCH = EC // CH
NG = CH // L
CPS = CH // NS

_mesh = plsc.VectorSubcoreMesh(core_axis_name="c", subcore_axis_name="s")


@functools.partial(
    pl.kernel,
    out_type=jax.ShapeDtypeStruct((NC, NCH, CH), jnp.float32),
    mesh=_mesh,
    scratch_types=[
        pltpu.VMEM((N_NODES, WPB), jnp.int32),
        pltpu.VMEM((2, CH), jnp.int32),
        pltpu.VMEM((2, CH), jnp.int32),
        pltpu.VMEM((CH,), jnp.float32),
        pltpu.VMEM((NS, CPS), jnp.float32),
        pltpu.VMEM((CPS,), jnp.float32),
        pltpu.VMEM_SHARED((2, NS, CH), jnp.float32),
        pltpu.SemaphoreType.DMA,
        pltpu.SemaphoreType.DMA,
    ],
    compiler_params=pltpu.CompilerParams(
        use_tc_tiling_on_sc=False, needs_layout_passes=False
    ),
)
def _sc_scores(
    hpk_hbm, u_hbm, v_hbm, out_hbm,
    tab, uv, vv, ps, rbuf, res, pbuf, si0, si1
):
    cid = lax.axis_index("c")
    sid = lax.axis_index("s")
    sis = (si0, si1)

    pltpu.sync_copy(hpk_hbm.at[sid], tab)

    def issue_idx(k, b):
        base = cid * EC + k * CH
        pltpu.async_copy(u_hbm.at[pl.ds(base, CH)], uv.at[b], sis[b])
        pltpu.async_copy(v_hbm.at[pl.ds(base, CH)], vv.at[b], sis[b])

    def wait_idx(b):
        pltpu.make_async_copy(u_hbm.at[pl.ds(0, CH)], uv.at[b], sis[b]).wait()
        pltpu.make_async_copy(v_hbm.at[pl.ds(0, CH)], vv.at[b], sis[b]).wait()

    def compute(b):
        @plsc.parallel_loop(0, NG, unroll=2)
        def _(g):
            uvec = uv[b, pl.ds(g * L, L)]
            vvec = vv[b, pl.ds(g * L, L)]
            a0 = jnp.zeros((L,), jnp.float32)
            a1 = jnp.zeros((L,), jnp.float32)
            for w in range(WPB):
                ws = jnp.full((L,), w, jnp.int32)
                aw = plsc.load_gather(tab, [uvec, ws])
                bw = plsc.load_gather(tab, [vvec, ws])
                p = plsc.bitcast(aw, jnp.bfloat16) * plsc.bitcast(
                    bw, jnp.bfloat16
                )
                p0, p1 = plsc.unpack(p, format=plsc.PackFormat.INTERLEAVED)
                a0 = a0 + p0
                a1 = a1 + p1
            ps[pl.ds(g * L, L)] = a0 + a1

    def reduce_and_store(k, b):
        pltpu.sync_copy(pbuf.at[b, :, pl.ds(sid * CPS, CPS)], rbuf)

        @plsc.parallel_loop(0, CPS // L)
        def _(j):
            s = rbuf[0, pl.ds(j * L, L)]
            for t in range(1, NS):
                s = s + rbuf[t, pl.ds(j * L, L)]
            res[pl.ds(j * L, L)] = s

        pltpu.sync_copy(res, out_hbm.at[cid, k, pl.ds(sid * CPS, CPS)])

    issue_idx(0, 0)

    def outer(k2, carry):
        for b in range(2):
            k = k2 * 2 + b

            @pl.when(k + 1 < NCH)
            def _():
                issue_idx(k + 1, (b + 1) % 2)

            wait_idx(b)
            compute(b)
            pltpu.sync_copy(ps, pbuf.at[b, sid])
            plsc.subcore_barrier()
            reduce_and_store(k, b)
        return carry

    lax.fori_loop(0, NCH // 2, outer, 0)


def kernel(h, edge_index):
    ei = edge_index.astype(jnp.int32)
    h_blk = lax.bitcast_convert_type(
        h.astype(jnp.bfloat16)
        .reshape(N_NODES, NS, WPB, 2)
        .transpose(1, 0, 2, 3),
        jnp.int32,
    )
    pad = jnp.zeros((E_PAD - N_EDGES,), jnp.int32)
    u = jnp.concatenate([ei[0], pad])
    v = jnp.concatenate([ei[1], pad])
    scores = _sc_scores(h_blk, u, v)
    return scores.reshape(-1)[:N_EDGES]


# speedup vs baseline: 1.0008x; 1.0008x over previous
import functools

import jax
import jax.numpy as jnp
from jax import lax
from jax.experimental import pallas as pl
from jax.experimental.pallas import tpu as pltpu
from jax.experimental.pallas import tpu_sc as plsc

N_NODES = 10000
N_EDGES = 160000
D_FEAT = 256
DW = D_FEAT // 2

NC, NS, L = 2, 16, 16
WPB = DW // NS
E_PAD = 163840
EC = E_PAD // NC
CH = 4096
NCH = EC // CH
NG = CH // L
CPS = CH // NS

_mesh = plsc.VectorSubcoreMesh(core_axis_name="c", subcore_axis_name="s")


@functools.partial(
    pl.kernel,
    out_type=jax.ShapeDtypeStruct((NC, NCH, CH), jnp.float32),
    mesh=_mesh,
    scratch_types=[
        pltpu.VMEM((N_NODES, WPB), jnp.int32),
        pltpu.VMEM((2, CH), jnp.int32),
        pltpu.VMEM((2, CH), jnp.int32),
        pltpu.VMEM((CH,), jnp.float32),
        pltpu.VMEM((NS, CPS), jnp.float32),
        pltpu.VMEM((CPS,), jnp.float32),
        pltpu.VMEM_SHARED((2, NS, CH), jnp.float32),
        pltpu.SemaphoreType.DMA,
        pltpu.SemaphoreType.DMA,
    ],
    compiler_params=pltpu.CompilerParams(
        use_tc_tiling_on_sc=False, needs_layout_passes=False
    ),
)
def _sc_scores(
    hpk_hbm, u_hbm, v_hbm, out_hbm,
    tab, uv, vv, ps, rbuf, res, pbuf, si0, si1
):
    cid = lax.axis_index("c")
    sid = lax.axis_index("s")
    sis = (si0, si1)

    pltpu.sync_copy(hpk_hbm.at[sid], tab)

    def issue_idx(k, b):
        base = cid * EC + k * CH
        pltpu.async_copy(u_hbm.at[pl.ds(base, CH)], uv.at[b], sis[b])
        pltpu.async_copy(v_hbm.at[pl.ds(base, CH)], vv.at[b], sis[b])

    def wait_idx(b):
        pltpu.make_async_copy(u_hbm.at[pl.ds(0, CH)], uv.at[b], sis[b]).wait()
        pltpu.make_async_copy(v_hbm.at[pl.ds(0, CH)], vv.at[b], sis[b]).wait()

    def compute(b):
        @plsc.parallel_loop(0, NG, unroll=2)
        def _(g):
            uvec = uv[b, pl.ds(g * L, L)]
            vvec = vv[b, pl.ds(g * L, L)]
            a0 = jnp.zeros((L,), jnp.float32)
            a1 = jnp.zeros((L,), jnp.float32)
            for w in range(WPB):
                ws = jnp.full((L,), w, jnp.int32)
                aw = plsc.load_gather(tab, [uvec, ws])
                bw = plsc.load_gather(tab, [vvec, ws])
                p = plsc.bitcast(aw, jnp.bfloat16) * plsc.bitcast(
                    bw, jnp.bfloat16
                )
                p0, p1 = plsc.unpack(p, format=plsc.PackFormat.INTERLEAVED)
                a0 = a0 + p0
                a1 = a1 + p1
            ps[pl.ds(g * L, L)] = a0 + a1

    def reduce_and_store(k, b):
        pltpu.sync_copy(pbuf.at[b, :, pl.ds(sid * CPS, CPS)], rbuf)

        @plsc.parallel_loop(0, CPS // L)
        def _(j):
            s = rbuf[0, pl.ds(j * L, L)]
            for t in range(1, NS):
                s = s + rbuf[t, pl.ds(j * L, L)]
            res[pl.ds(j * L, L)] = s

        pltpu.sync_copy(res, out_hbm.at[cid, k, pl.ds(sid * CPS, CPS)])

    issue_idx(0, 0)

    def outer(k2, carry):
        for b in range(2):
            k = k2 * 2 + b

            @pl.when(k + 1 < NCH)
            def _():
                issue_idx(k + 1, (b + 1) % 2)

            with jax.named_scope("idxw"):
                wait_idx(b)
            with jax.named_scope("comp"):
                compute(b)
            with jax.named_scope("pwr"):
                pltpu.sync_copy(ps, pbuf.at[b, sid])
            with jax.named_scope("bar"):
                plsc.subcore_barrier()
            with jax.named_scope("red"):
                reduce_and_store(k, b)
        return carry

    lax.fori_loop(0, NCH // 2, outer, 0)


def kernel(h, edge_index):
    ei = edge_index.astype(jnp.int32)
    h_blk = lax.bitcast_convert_type(
        h.astype(jnp.bfloat16)
        .reshape(N_NODES, NS, WPB, 2)
        .transpose(1, 0, 2, 3),
        jnp.int32,
    )
    pad = jnp.zeros((E_PAD - N_EDGES,), jnp.int32)
    u = jnp.concatenate([ei[0], pad])
    v = jnp.concatenate([ei[1], pad])
    scores = _sc_scores(h_blk, u, v)
    return scores.reshape(-1)[:N_EDGES]


# speedup vs baseline: 1.6994x; 1.6980x over previous
import functools

import jax
import jax.numpy as jnp
from jax import lax
from jax.experimental import pallas as pl
from jax.experimental.pallas import tpu as pltpu
from jax.experimental.pallas import tpu_sc as plsc

N_NODES = 10000
N_EDGES = 160000
D_FEAT = 256
DW = D_FEAT // 2

NC, NS, L = 2, 16, 16
WPB = DW // NS
E_PAD = 163840
EC = E_PAD // NC
CH = 4096
NCH = EC // CH
NG = CH // L
CPS = CH // NS

_mesh = plsc.VectorSubcoreMesh(core_axis_name="c", subcore_axis_name="s")


@functools.partial(
    pl.kernel,
    out_type=jax.ShapeDtypeStruct((NC, NCH, CH), jnp.float32),
    mesh=_mesh,
    scratch_types=[
        pltpu.VMEM((WPB, N_NODES), jnp.int32),
        pltpu.VMEM((2, CH), jnp.int32),
        pltpu.VMEM((2, CH), jnp.int32),
        pltpu.VMEM((CH,), jnp.float32),
        pltpu.VMEM((NS, CPS), jnp.float32),
        pltpu.VMEM((CPS,), jnp.float32),
        pltpu.VMEM_SHARED((2, NS, CH), jnp.float32),
        pltpu.SemaphoreType.DMA,
        pltpu.SemaphoreType.DMA,
    ],
    compiler_params=pltpu.CompilerParams(
        use_tc_tiling_on_sc=False, needs_layout_passes=False
    ),
)
def _sc_scores(
    hpk_hbm, u_hbm, v_hbm, out_hbm,
    tab, uv, vv, ps, rbuf, res, pbuf, si0, si1
):
    cid = lax.axis_index("c")
    sid = lax.axis_index("s")
    sis = (si0, si1)

    pltpu.sync_copy(hpk_hbm.at[sid], tab)

    def issue_idx(k, b):
        base = cid * EC + k * CH
        pltpu.async_copy(u_hbm.at[pl.ds(base, CH)], uv.at[b], sis[b])
        pltpu.async_copy(v_hbm.at[pl.ds(base, CH)], vv.at[b], sis[b])

    def wait_idx(b):
        pltpu.make_async_copy(u_hbm.at[pl.ds(0, CH)], uv.at[b], sis[b]).wait()
        pltpu.make_async_copy(v_hbm.at[pl.ds(0, CH)], vv.at[b], sis[b]).wait()

    def compute(b):
        @plsc.parallel_loop(0, NG, unroll=2)
        def _(g):
            uvec = uv[b, pl.ds(g * L, L)]
            vvec = vv[b, pl.ds(g * L, L)]
            a0 = jnp.zeros((L,), jnp.float32)
            a1 = jnp.zeros((L,), jnp.float32)
            for w in range(WPB):
                ws = jnp.full((L,), w, jnp.int32)
                aw = plsc.load_gather(tab, [ws, uvec])
                bw = plsc.load_gather(tab, [ws, vvec])
                p = plsc.bitcast(aw, jnp.bfloat16) * plsc.bitcast(
                    bw, jnp.bfloat16
                )
                p0, p1 = plsc.unpack(p, format=plsc.PackFormat.INTERLEAVED)
                a0 = a0 + p0
                a1 = a1 + p1
            ps[pl.ds(g * L, L)] = a0 + a1

    def reduce_and_store(k, b):
        pltpu.sync_copy(pbuf.at[b, :, pl.ds(sid * CPS, CPS)], rbuf)

        @plsc.parallel_loop(0, CPS // L)
        def _(j):
            s = rbuf[0, pl.ds(j * L, L)]
            for t in range(1, NS):
                s = s + rbuf[t, pl.ds(j * L, L)]
            res[pl.ds(j * L, L)] = s

        pltpu.sync_copy(res, out_hbm.at[cid, k, pl.ds(sid * CPS, CPS)])

    issue_idx(0, 0)

    def outer(k2, carry):
        for b in range(2):
            k = k2 * 2 + b

            @pl.when(k + 1 < NCH)
            def _():
                issue_idx(k + 1, (b + 1) % 2)

            wait_idx(b)
            compute(b)
            pltpu.sync_copy(ps, pbuf.at[b, sid])
            plsc.subcore_barrier()
            reduce_and_store(k, b)
        return carry

    lax.fori_loop(0, NCH // 2, outer, 0)


def kernel(h, edge_index):
    ei = edge_index.astype(jnp.int32)
    h_pk = lax.bitcast_convert_type(
        h.astype(jnp.bfloat16).reshape(N_NODES, DW, 2), jnp.int32
    )
    h_blk = h_pk.T.reshape(NS, WPB, N_NODES)
    pad = jnp.zeros((E_PAD - N_EDGES,), jnp.int32)
    u = jnp.concatenate([ei[0], pad])
    v = jnp.concatenate([ei[1], pad])
    scores = _sc_scores(h_blk, u, v)
    return scores.reshape(-1)[:N_EDGES]
